# Initial kernel scaffold; baseline (speedup 1.0000x reference)
import jax
import jax.numpy as jnp
from jax.experimental import pallas as pl


def kernel(token_ids, segment_ids, token_table, position_table, segment_table, gamma, beta):
    raise NotImplementedError("write your pallas kernel here")


# speedup vs baseline: 7.5608x; 7.5608x over previous
import functools

import jax
import jax.numpy as jnp
from jax import lax
from jax.experimental import pallas as pl
from jax.experimental.pallas import tpu as pltpu
from jax.experimental.pallas import tpu_sc as plsc

_EMBED = 128


def _sc_gather(table, idx_flat):
    n = idx_flat.shape[0]
    info = plsc.get_sparse_core_info()
    nw = info.num_cores * info.num_subcores
    per_w = n // nw
    ch = 640 if per_w % 640 == 0 else per_w
    n_ch = per_w // ch

    @functools.partial(
        pl.kernel,
        mesh=plsc.VectorSubcoreMesh(core_axis_name="c", subcore_axis_name="s"),
        out_type=jax.ShapeDtypeStruct((n, _EMBED), jnp.float32),
        scratch_types=[
            pltpu.VMEM((ch,), jnp.int32),
            pltpu.VMEM((ch, _EMBED), jnp.float32),
            pltpu.SemaphoreType.DMA,
        ],
    )
    def k(table_hbm, idx_hbm, out_hbm, idx_v, rows_v, sem):
        wid = lax.axis_index("s") * info.num_cores + lax.axis_index("c")
        base = wid * per_w

        def body(i, carry):
            off = base + i * ch
            pltpu.sync_copy(idx_hbm.at[pl.ds(off, ch)], idx_v)
            pltpu.async_copy(table_hbm.at[idx_v], rows_v, sem).wait()
            pltpu.sync_copy(rows_v, out_hbm.at[pl.ds(off, ch)])
            return carry

        lax.fori_loop(0, n_ch, body, 0)

    return k(table, idx_flat)


def _tc_body(tok_ref, segf_ref, pos_ref, segtab_ref, g_ref, b_ref, o_ref):
    x = tok_ref[...]
    s = x.shape[1]
    pos = pos_ref[:s, :][None, :, :]
    segf = segf_ref[...]
    s0 = segtab_ref[0:1, :][None, :, :]
    s1 = segtab_ref[1:2, :][None, :, :]
    x = x + pos + s0 + segf * (s1 - s0)
    mean = jnp.mean(x, axis=-1, keepdims=True)
    var = jnp.mean((x - mean) ** 2, axis=-1, keepdims=True)
    xn = (x - mean) * lax.rsqrt(var + 1e-5)
    o_ref[...] = xn * g_ref[0:1, :][None, :, :] + b_ref[0:1, :][None, :, :]


def _tc_layernorm(tok_rows, segment_ids, position_table, segment_table, gamma, beta):
    b, s, _ = tok_rows.shape
    bb = 8
    grid = (b // bb,)
    segf = segment_ids.astype(jnp.float32).reshape(b, s, 1)
    return pl.pallas_call(
        _tc_body,
        grid=grid,
        in_specs=[
            pl.BlockSpec((bb, s, _EMBED), lambda i: (i, 0, 0)),
            pl.BlockSpec((bb, s, 1), lambda i: (i, 0, 0)),
            pl.BlockSpec(position_table.shape, lambda i: (0, 0)),
            pl.BlockSpec(segment_table.shape, lambda i: (0, 0)),
            pl.BlockSpec((1, _EMBED), lambda i: (0, 0)),
            pl.BlockSpec((1, _EMBED), lambda i: (0, 0)),
        ],
        out_specs=pl.BlockSpec((bb, s, _EMBED), lambda i: (i, 0, 0)),
        out_shape=jax.ShapeDtypeStruct((b, s, _EMBED), jnp.float32),
    )(tok_rows, segf, position_table, segment_table,
      gamma.reshape(1, _EMBED), beta.reshape(1, _EMBED))


def kernel(token_ids, segment_ids, token_table, position_table, segment_table, gamma, beta):
    b, s = token_ids.shape
    idx_flat = token_ids.reshape(-1).astype(jnp.int32)
    rows = _sc_gather(token_table, idx_flat)
    return _tc_layernorm(rows.reshape(b, s, _EMBED),
                         segment_ids.astype(jnp.int32),
                         position_table, segment_table, gamma, beta)


# speedup vs baseline: 7.6757x; 1.0152x over previous
import functools

import jax
import jax.numpy as jnp
from jax import lax
from jax.experimental import pallas as pl
from jax.experimental.pallas import tpu as pltpu
from jax.experimental.pallas import tpu_sc as plsc

_EMBED = 128


def _sc_gather(table, idx_flat):
    n = idx_flat.shape[0]
    info = plsc.get_sparse_core_info()
    nw = info.num_cores * info.num_subcores
    per_w = n // nw
    ch = 400
    n_ch = per_w // ch

    @functools.partial(
        pl.kernel,
        mesh=plsc.VectorSubcoreMesh(core_axis_name="c", subcore_axis_name="s"),
        out_type=jax.ShapeDtypeStruct((n, _EMBED), jnp.float32),
        scratch_types=[
            pltpu.VMEM((per_w,), jnp.int32),
            pltpu.VMEM((2, ch, _EMBED), jnp.float32),
            pltpu.SemaphoreType.DMA,
            pltpu.SemaphoreType.DMA,
            pltpu.SemaphoreType.DMA,
            pltpu.SemaphoreType.DMA,
        ],
    )
    def k(table_hbm, idx_hbm, out_hbm, idx_v, rows_v, gs0, gs1, os0, os1):
        wid = lax.axis_index("s") * info.num_cores + lax.axis_index("c")
        base = wid * per_w
        gsem = (gs0, gs1)
        osem = (os0, os1)
        pltpu.sync_copy(idx_hbm.at[pl.ds(base, per_w)], idx_v)
        gh = [None, None]
        oh = [None, None]
        gh[0] = pltpu.async_copy(
            table_hbm.at[idx_v.at[pl.ds(0, ch)]], rows_v.at[0], gsem[0])
        for i in range(n_ch):
            b = i % 2
            nb = (i + 1) % 2
            if i + 1 < n_ch:
                if i >= 1:
                    oh[nb].wait()
                gh[nb] = pltpu.async_copy(
                    table_hbm.at[idx_v.at[pl.ds((i + 1) * ch, ch)]],
                    rows_v.at[nb], gsem[nb])
            gh[b].wait()
            oh[b] = pltpu.async_copy(
                rows_v.at[b], out_hbm.at[pl.ds(base + i * ch, ch)], osem[b])
        oh[0].wait()
        oh[1].wait()

    return k(table, idx_flat)


def _tc_body(tok_ref, segf_ref, pos_ref, segtab_ref, g_ref, b_ref, o_ref):
    x = tok_ref[...]
    s = x.shape[1]
    pos = pos_ref[:s, :][None, :, :]
    segf = segf_ref[...]
    s0 = segtab_ref[0:1, :][None, :, :]
    s1 = segtab_ref[1:2, :][None, :, :]
    x = x + pos + s0 + segf * (s1 - s0)
    mean = jnp.mean(x, axis=-1, keepdims=True)
    var = jnp.mean((x - mean) ** 2, axis=-1, keepdims=True)
    xn = (x - mean) * lax.rsqrt(var + 1e-5)
    o_ref[...] = xn * g_ref[0:1, :][None, :, :] + b_ref[0:1, :][None, :, :]


def _tc_layernorm(tok_rows, segment_ids, position_table, segment_table, gamma, beta):
    b, s, _ = tok_rows.shape
    bb = 8
    grid = (b // bb,)
    segf = segment_ids.astype(jnp.float32).reshape(b, s, 1)
    return pl.pallas_call(
        _tc_body,
        grid=grid,
        in_specs=[
            pl.BlockSpec((bb, s, _EMBED), lambda i: (i, 0, 0)),
            pl.BlockSpec((bb, s, 1), lambda i: (i, 0, 0)),
            pl.BlockSpec(position_table.shape, lambda i: (0, 0)),
            pl.BlockSpec(segment_table.shape, lambda i: (0, 0)),
            pl.BlockSpec((1, _EMBED), lambda i: (0, 0)),
            pl.BlockSpec((1, _EMBED), lambda i: (0, 0)),
        ],
        out_specs=pl.BlockSpec((bb, s, _EMBED), lambda i: (i, 0, 0)),
        out_shape=jax.ShapeDtypeStruct((b, s, _EMBED), jnp.float32),
    )(tok_rows, segf, position_table, segment_table,
      gamma.reshape(1, _EMBED), beta.reshape(1, _EMBED))


def kernel(token_ids, segment_ids, token_table, position_table, segment_table, gamma, beta):
    b, s = token_ids.shape
    idx_flat = token_ids.reshape(-1).astype(jnp.int32)
    rows = _sc_gather(token_table, idx_flat)
    return _tc_layernorm(rows.reshape(b, s, _EMBED),
                         segment_ids.astype(jnp.int32),
                         position_table, segment_table, gamma, beta)


# speedup vs baseline: 7.8944x; 1.0285x over previous
import functools

import jax
import jax.numpy as jnp
from jax import lax
from jax.experimental import pallas as pl
from jax.experimental.pallas import tpu as pltpu
from jax.experimental.pallas import tpu_sc as plsc

_EMBED = 128


def _sc_gather(table, idx_flat):
    n = idx_flat.shape[0]
    info = plsc.get_sparse_core_info()
    nw = info.num_cores * info.num_subcores
    per_w = n // nw
    ch = 400
    n_ch = per_w // ch

    @functools.partial(
        pl.kernel,
        mesh=plsc.VectorSubcoreMesh(core_axis_name="c", subcore_axis_name="s"),
        out_type=jax.ShapeDtypeStruct((n, _EMBED), jnp.float32),
        scratch_types=[
            pltpu.VMEM((per_w,), jnp.int32),
            pltpu.VMEM((2, ch, _EMBED), jnp.float32),
            pltpu.SemaphoreType.DMA,
            pltpu.SemaphoreType.DMA,
            pltpu.SemaphoreType.DMA,
            pltpu.SemaphoreType.DMA,
        ],
    )
    def k(table_hbm, idx_hbm, out_hbm, idx_v, rows_v, gs0, gs1, os0, os1):
        wid = lax.axis_index("s") * info.num_cores + lax.axis_index("c")
        base = wid * per_w
        gsem = (gs0, gs1)
        osem = (os0, os1)
        pltpu.sync_copy(idx_hbm.at[pl.ds(base, per_w)], idx_v)
        gh = [None, None]
        oh = [None, None]
        gh[0] = pltpu.async_copy(
            table_hbm.at[idx_v.at[pl.ds(0, ch)]], rows_v.at[0], gsem[0])
        for i in range(n_ch):
            b = i % 2
            nb = (i + 1) % 2
            if i + 1 < n_ch:
                if i >= 1:
                    oh[nb].wait()
                gh[nb] = pltpu.async_copy(
                    table_hbm.at[idx_v.at[pl.ds((i + 1) * ch, ch)]],
                    rows_v.at[nb], gsem[nb])
            gh[b].wait()
            oh[b] = pltpu.async_copy(
                rows_v.at[b], out_hbm.at[pl.ds(base + i * ch, ch)], osem[b])
        oh[0].wait()
        oh[1].wait()

    return k(table, idx_flat)


def _tc_body(tok_ref, segf_ref, pos_ref, segtab_ref, g_ref, b_ref, o_ref):
    x = tok_ref[...]
    segf = segf_ref[...]
    s0 = segtab_ref[0:1, :]
    s1 = segtab_ref[1:2, :]
    x = x + pos_ref[...] + s0 + segf * (s1 - s0)
    ones = jnp.ones((_EMBED, _EMBED), jnp.float32)
    mean = jax.lax.dot(x, ones) * (1.0 / _EMBED)
    xc = x - mean
    var = jax.lax.dot(xc * xc, ones) * (1.0 / _EMBED)
    xn = xc * lax.rsqrt(var + 1e-5)
    o_ref[...] = xn * g_ref[0:1, :] + b_ref[0:1, :]


def _tc_layernorm(tok_rows, segment_ids, position_table, segment_table, gamma, beta):
    b, s, _ = tok_rows.shape
    n = b * s
    bl = 8 * s
    grid = (n // bl,)
    segf = segment_ids.astype(jnp.float32).reshape(n, 1)
    rows2d = tok_rows.reshape(n, _EMBED)
    pos_big = jnp.tile(position_table[:s, :], (bl // s, 1))
    out = pl.pallas_call(
        _tc_body,
        grid=grid,
        in_specs=[
            pl.BlockSpec((bl, _EMBED), lambda i: (i, 0)),
            pl.BlockSpec((bl, 1), lambda i: (i, 0)),
            pl.BlockSpec((bl, _EMBED), lambda i: (0, 0)),
            pl.BlockSpec(segment_table.shape, lambda i: (0, 0)),
            pl.BlockSpec((1, _EMBED), lambda i: (0, 0)),
            pl.BlockSpec((1, _EMBED), lambda i: (0, 0)),
        ],
        out_specs=pl.BlockSpec((bl, _EMBED), lambda i: (i, 0)),
        out_shape=jax.ShapeDtypeStruct((n, _EMBED), jnp.float32),
    )(rows2d, segf, pos_big, segment_table,
      gamma.reshape(1, _EMBED), beta.reshape(1, _EMBED))
    return out.reshape(b, s, _EMBED)


def kernel(token_ids, segment_ids, token_table, position_table, segment_table, gamma, beta):
    b, s = token_ids.shape
    idx_flat = token_ids.reshape(-1).astype(jnp.int32)
    rows = _sc_gather(token_table, idx_flat)
    return _tc_layernorm(rows.reshape(b, s, _EMBED),
                         segment_ids.astype(jnp.int32),
                         position_table, segment_table, gamma, beta)
